# baseline (speedup 1.0000x reference)
import functools

import jax
import jax.numpy as jnp
from jax import lax
from jax.experimental import pallas as pl
from jax.experimental.pallas import tpu as pltpu
from jax.experimental.pallas import tpu_sc as plsc

N_ENTITY = 1000000
N_RELATION = 1000
HIDDEN = 64
BATCH = 4096

_NC, _NS = 2, 16
_NW = _NC * _NS
_BW = BATCH // _NW

_mesh = plsc.VectorSubcoreMesh(core_axis_name="c", subcore_axis_name="s")


@functools.partial(
    pl.kernel,
    mesh=_mesh,
    compiler_params=pltpu.CompilerParams(use_tc_tiling_on_sc=False),
    out_type=(
        jax.ShapeDtypeStruct((BATCH, HIDDEN), jnp.float32),
        jax.ShapeDtypeStruct((BATCH, HIDDEN), jnp.float32),
        jax.ShapeDtypeStruct((BATCH, HIDDEN), jnp.float32),
        jax.ShapeDtypeStruct((BATCH, HIDDEN), jnp.float32),
        jax.ShapeDtypeStruct((BATCH,), jnp.float32),
        jax.ShapeDtypeStruct((BATCH, 2 * HIDDEN), jnp.float32),
        jax.ShapeDtypeStruct((BATCH, HIDDEN), jnp.float32),
        jax.ShapeDtypeStruct((BATCH,), jnp.float32),
        jax.ShapeDtypeStruct((BATCH,), jnp.float32),
        jax.ShapeDtypeStruct((BATCH,), jnp.float32),
    ),
    scratch_types=(
        pltpu.VMEM((_BW,), jnp.int32),
        pltpu.VMEM((_BW,), jnp.int32),
        pltpu.VMEM((_BW,), jnp.int32),
        pltpu.VMEM((_BW,), jnp.int32),
        pltpu.VMEM((_BW, HIDDEN), jnp.float32),
        pltpu.VMEM((_BW, HIDDEN), jnp.float32),
        pltpu.VMEM((_BW, HIDDEN), jnp.float32),
        pltpu.VMEM((_BW, HIDDEN), jnp.float32),
        pltpu.VMEM((_BW,), jnp.float32),
        pltpu.VMEM((_BW, 2 * HIDDEN), jnp.float32),
        pltpu.VMEM((_BW, HIDDEN), jnp.float32),
        pltpu.VMEM((_BW,), jnp.float32),
        pltpu.VMEM((_BW,), jnp.float32),
        pltpu.VMEM((_BW,), jnp.float32),
        pltpu.SemaphoreType.DMA,
        pltpu.SemaphoreType.DMA,
    ),
)
def _gather_all(entity_emb, relation_emb, relation_diag, curvature, context,
                head_bias, tail_bias, head, tail, rel, neg,
                head_o, tail_o, rel_o, neg_o, curv_o, diag_o, ctx_o,
                hb_o, tb_o, ntb_o,
                hidx, tidx, ridx, nidx,
                hrow, trow, rrow, nrow, crow, drow, xrow, hbrow, tbrow, ntbrow,
                gsem, ssem):
    wid = lax.axis_index("s") * _NC + lax.axis_index("c")
    base = wid * _BW
    sl = pl.ds(base, _BW)

    pltpu.sync_copy(head.at[sl], hidx)
    pltpu.sync_copy(tail.at[sl], tidx)
    pltpu.sync_copy(rel.at[sl], ridx)
    pltpu.sync_copy(neg.at[sl], nidx)

    gathers = [
        pltpu.async_copy(entity_emb.at[hidx], hrow, gsem),
        pltpu.async_copy(entity_emb.at[tidx], trow, gsem),
        pltpu.async_copy(relation_emb.at[ridx], rrow, gsem),
        pltpu.async_copy(entity_emb.at[nidx], nrow, gsem),
        pltpu.async_copy(curvature.at[ridx], crow, gsem),
        pltpu.async_copy(relation_diag.at[ridx], drow, gsem),
        pltpu.async_copy(context.at[ridx], xrow, gsem),
        pltpu.async_copy(head_bias.at[hidx], hbrow, gsem),
        pltpu.async_copy(tail_bias.at[tidx], tbrow, gsem),
        pltpu.async_copy(tail_bias.at[nidx], ntbrow, gsem),
    ]
    for g in gathers:
        g.wait()

    stores = [
        pltpu.async_copy(hrow, head_o.at[sl], ssem),
        pltpu.async_copy(trow, tail_o.at[sl], ssem),
        pltpu.async_copy(rrow, rel_o.at[sl], ssem),
        pltpu.async_copy(nrow, neg_o.at[sl], ssem),
        pltpu.async_copy(crow, curv_o.at[sl], ssem),
        pltpu.async_copy(drow, diag_o.at[sl], ssem),
        pltpu.async_copy(xrow, ctx_o.at[sl], ssem),
        pltpu.async_copy(hbrow, hb_o.at[sl], ssem),
        pltpu.async_copy(tbrow, tb_o.at[sl], ssem),
        pltpu.async_copy(ntbrow, ntb_o.at[sl], ssem),
    ]
    for s in stores:
        s.wait()


def kernel(entity_emb, relation_emb, relation_diag, curvature, context,
           head_bias, tail_bias, head, tail, rel, neg):
    scale = jnp.array([0.125], dtype=jnp.float32)
    (head_e, tail_e, rel_e, neg_e, curv, rel_diag, ctx,
     h_bias, t_bias, neg_t_bias) = _gather_all(
        entity_emb, relation_emb, relation_diag,
        curvature.reshape(N_RELATION), context,
        head_bias.reshape(N_ENTITY), tail_bias.reshape(N_ENTITY),
        head.astype(jnp.int32), tail.astype(jnp.int32),
        rel.astype(jnp.int32), neg.astype(jnp.int32))
    return (scale, head_e, tail_e, rel_e, neg_e,
            curv.reshape(BATCH, 1), rel_diag, ctx,
            h_bias.reshape(BATCH, 1), t_bias.reshape(BATCH, 1),
            neg_t_bias.reshape(BATCH, 1))
